# baseline (speedup 1.0000x reference)
import functools

import jax
import jax.numpy as jnp
from jax import lax
from jax.experimental import pallas as pl
from jax.experimental.pallas import tpu as pltpu
from jax.experimental.pallas import tpu_sc as plsc

N, C, B, CTX = 100000, 256, 4, 256
NC, NS, L = 2, 16, 16
NW = NC * NS
CH = 120
G = C // L
BBUF = CH + L

N_SC = 36000
FULLS = N_SC // CH
NPAIRS = FULLS // (2 * NW)
LEFTS = FULLS - 2 * NPAIRS * NW

RB = 2000
OFF = N_SC // RB
NBT = (N - N_SC) // RB

FULLA = N // CH
NPAIRA = FULLA // NW // 2
TAILA = N - FULLA * CH

_mesh = plsc.VectorSubcoreMesh(core_axis_name="c", subcore_axis_name="s",
                               num_cores=NC, num_subcores=NS)


def _wid():
    return lax.axis_index("s") * NC + lax.axis_index("c")


def _row0(k):
    return pl.multiple_of(k * CH, 8)


def _accum_carry(xbuf, carry, nrows):
    def row_body(r, c):
        out = []
        for g in range(G):
            v = xbuf[r, pl.ds(g * L, L)]
            out.append(c[g] + v)
        for g in range(G):
            v = xbuf[r, pl.ds(g * L, L)]
            out.append(c[G + g] + v * v)
        return tuple(out)

    return lax.fori_loop(0, nrows, row_body, carry)


def _accum_ref(xbuf, accbuf, nrows):
    def row_body(r, _):
        for g in range(G):
            v = xbuf[r, pl.ds(g * L, L)]
            accbuf[0, pl.ds(g * L, L)] += v
            accbuf[1, pl.ds(g * L, L)] += v * v
        return 0

    lax.fori_loop(0, nrows, row_body, 0)


def _stats_body(x_hbm, out_hbm, xa, xb, accbuf, redbuf, shared,
                sem_a, sem_b):
    wid = _wid()
    has_odd_a = wid + 2 * NPAIRS * NW < FULLS
    has_odd_b = wid + (2 * NPAIRS + 1) * NW < FULLS

    def start(k, buf, sem):
        pltpu.async_copy(x_hbm.at[pl.ds(_row0(k), CH), :], buf, sem)

    def wait(buf, sem):
        pltpu.make_async_copy(x_hbm.at[pl.ds(0, CH), :], buf, sem).wait()

    start(wid, xa, sem_a)
    start(wid + NW, xb, sem_b)

    def body(i, carry):
        wait(xa, sem_a)
        carry = _accum_carry(xa, carry, CH)

        @pl.when((i < NPAIRS - 1) | has_odd_a)
        def _():
            start(wid + (2 * i + 2) * NW, xa, sem_a)

        wait(xb, sem_b)
        carry = _accum_carry(xb, carry, CH)

        @pl.when((i < NPAIRS - 1) | has_odd_b)
        def _():
            start(wid + (2 * i + 3) * NW, xb, sem_b)

        return carry

    zero = jnp.zeros((L,), jnp.float32)
    acc = lax.fori_loop(0, NPAIRS, body, (zero,) * (2 * G))

    for g in range(G):
        accbuf[0, pl.ds(g * L, L)] = acc[g]
        accbuf[1, pl.ds(g * L, L)] = acc[G + g]

    @pl.when(has_odd_a)
    def _():
        wait(xa, sem_a)
        _accum_ref(xa, accbuf, CH)

    @pl.when(has_odd_b)
    def _():
        wait(xb, sem_b)
        _accum_ref(xb, accbuf, CH)

    sid = lax.axis_index("s")
    cid = lax.axis_index("c")
    pltpu.sync_copy(accbuf, shared.at[sid])
    plsc.subcore_barrier()

    @pl.when(sid == 0)
    def _():
        pltpu.sync_copy(shared, redbuf)

        def red_body(s, c):
            out = []
            for g in range(G):
                out.append(c[g] + redbuf[s, 0, pl.ds(g * L, L)])
            for g in range(G):
                out.append(c[G + g] + redbuf[s, 1, pl.ds(g * L, L)])
            return tuple(out)

        acc2 = lax.fori_loop(0, NS, red_body, (zero,) * (2 * G))
        for g in range(G):
            accbuf[0, pl.ds(g * L, L)] = acc2[g]
            accbuf[1, pl.ds(g * L, L)] = acc2[G + g]
        pltpu.sync_copy(accbuf, out_hbm.at[pl.ds(2 * cid, 2), :])


_sc_stats = functools.partial(
    pl.kernel,
    out_type=jax.ShapeDtypeStruct((2 * NC, C), jnp.float32),
    mesh=_mesh,
    scratch_types=[
        pltpu.VMEM((CH, C), jnp.float32),
        pltpu.VMEM((CH, C), jnp.float32),
        pltpu.VMEM((2, C), jnp.float32),
        pltpu.VMEM((NS, 2, C), jnp.float32),
        pltpu.VMEM_SHARED((NS, 2, C), jnp.float32),
        pltpu.SemaphoreType.DMA,
        pltpu.SemaphoreType.DMA,
    ],
)(_stats_body)


def _tc_stats_body(tok_ref, w_ref, b_ref, x_ref, sums_ref, sc_ref):
    @pl.when(pl.program_id(0) == 0)
    def _():
        sums_ref[...] = jnp.zeros_like(sums_ref)
        tok = tok_ref[...]
        h = tok * (1.0 / (1.0 + jnp.exp(-tok)))
        sc = lax.dot_general(h, w_ref[...], (((1,), (1,)), ((), ())),
                             preferred_element_type=jnp.float32)
        sc_ref[...] = sc + b_ref[...]

    xv = x_ref[...]
    sums_ref[0:1, :] += jnp.sum(xv, axis=0, keepdims=True)
    sums_ref[1:2, :] += jnp.sum(xv * xv, axis=0, keepdims=True)


def _rsqrt16(v):
    t = v
    r = jnp.full((L,), 1.0, jnp.float32)
    for k in (32, 16, 8, 4, 2, 1):
        p4 = float(4.0 ** k)
        big = t >= p4
        t = jnp.where(big, t * (1.0 / p4), t)
        r = jnp.where(big, r * (1.0 / 2.0 ** k), r)
        small = t < 1.0 / p4
        t = jnp.where(small, t * p4, t)
        r = jnp.where(small, r * (2.0 ** k), r)
    y = jnp.full((L,), 0.7, jnp.float32)
    for _ in range(6):
        y = y * (1.5 - 0.5 * t * y * y)
    return r * y


def _build_ad(sums_hbm, tcsums_hbm, sc_hbm, gamma_hbm, beta_hbm,
              stage, scbuf, gbuf, betabuf, adbuf, sem):
    cps = [
        pltpu.async_copy(sums_hbm, stage.at[pl.ds(0, 2 * NC), :], sem),
        pltpu.async_copy(tcsums_hbm.at[pl.ds(0, 2), :],
                         stage.at[pl.ds(2 * NC, 2), :], sem),
        pltpu.async_copy(sc_hbm, scbuf, sem),
        pltpu.async_copy(gamma_hbm, gbuf, sem),
        pltpu.async_copy(beta_hbm, betabuf, sem),
    ]
    for cp in cps:
        cp.wait()

    inv_n = 1.0 / N
    for g in range(G):
        sl = pl.ds(g * L, L)
        mean = (stage[0, sl] + stage[2, sl] + stage[4, sl]) * inv_n
        var = ((stage[1, sl] + stage[3, sl] + stage[5, sl]) * inv_n
               - mean * mean)
        gs = gbuf[sl] * _rsqrt16(var + 1e-5)
        base_shift = betabuf[sl] - mean * gs
        for bb in range(B):
            shift = scbuf[bb, sl]
            scale = scbuf[bb, pl.ds(C + g * L, L)]
            one_p = 1.0 + scale
            adbuf[pl.ds(bb * C + g * L, L)] = one_p * gs
            adbuf[pl.ds(B * C + bb * C + g * L, L)] = (
                one_p * base_shift + shift)


def _gather_ad(adbuf, brow):
    base = brow * C
    avs, dvs = [], []
    for g in range(G):
        avs.append(adbuf[pl.ds(base + g * L, L)])
        dvs.append(adbuf[pl.ds(base + (B * C + g * L), L)])
    return avs, dvs


def _apply_chunk(inbuf, outbuf, bbuf, adbuf, nrows):
    b0 = bbuf[pl.ds(0, L)][0]
    b1 = bbuf[pl.ds(nrows - L, L)][L - 1]

    @pl.when(b0 == b1)
    def _():
        avs, dvs = _gather_ad(adbuf, b0)

        def row_body(r, _):
            for g in range(G):
                sl = pl.ds(g * L, L)
                outbuf[r, sl] = inbuf[r, sl] * avs[g] + dvs[g]
            return 0

        lax.fori_loop(0, nrows, row_body, 0)

    @pl.when(b0 != b1)
    def _():
        def row_body(r, _):
            avs, dvs = _gather_ad(adbuf, bbuf[pl.ds(r, L)][0])
            for g in range(G):
                sl = pl.ds(g * L, L)
                outbuf[r, sl] = inbuf[r, sl] * avs[g] + dvs[g]
            return 0

        lax.fori_loop(0, nrows, row_body, 0)


def _apply_body(x_hbm, bidx_hbm, sums_hbm, tcsums_hbm, sc_hbm,
                gamma_hbm, beta_hbm, out_hbm,
                ina, inb, outa, outb, bba, bbb, adbuf,
                stage, scbuf, gbuf, betabuf,
                sem_ia, sem_ib, sem_oa, sem_ob, sem_ba, sem_bb):
    wid = _wid()
    has_odd = wid + 2 * NPAIRA * NW < FULLA
    is_tail = wid == NW - 1

    def start_in(k, buf, bbuf, sem, bsem):
        r0 = _row0(k)
        pltpu.async_copy(x_hbm.at[pl.ds(r0, CH), :], buf, sem)
        pltpu.async_copy(bidx_hbm.at[pl.ds(r0, CH)],
                         bbuf.at[pl.ds(0, CH)], bsem)

    def wait_in(buf, bbuf, sem, bsem):
        pltpu.make_async_copy(x_hbm.at[pl.ds(0, CH), :], buf, sem).wait()
        pltpu.make_async_copy(bidx_hbm.at[pl.ds(0, CH)],
                              bbuf.at[pl.ds(0, CH)], bsem).wait()

    def start_out(k, buf, sem):
        pltpu.async_copy(buf, out_hbm.at[pl.ds(_row0(k), CH), :], sem)

    def wait_out(buf, sem):
        pltpu.make_async_copy(buf, out_hbm.at[pl.ds(0, CH), :], sem).wait()

    start_in(wid, ina, bba, sem_ia, sem_ba)
    start_in(wid + NW, inb, bbb, sem_ib, sem_bb)

    _build_ad(sums_hbm, tcsums_hbm, sc_hbm, gamma_hbm, beta_hbm,
              stage, scbuf, gbuf, betabuf, adbuf, sem_oa)

    def body(i, _):
        ka = wid + 2 * i * NW

        @pl.when(i > 0)
        def _():
            wait_out(outa, sem_oa)

        wait_in(ina, bba, sem_ia, sem_ba)
        _apply_chunk(ina, outa, bba, adbuf, CH)
        start_out(ka, outa, sem_oa)

        @pl.when((i < NPAIRA - 1) | has_odd)
        def _():
            start_in(ka + 2 * NW, ina, bba, sem_ia, sem_ba)

        kb = ka + NW

        @pl.when(i > 0)
        def _():
            wait_out(outb, sem_ob)

        wait_in(inb, bbb, sem_ib, sem_bb)
        _apply_chunk(inb, outb, bbb, adbuf, CH)
        start_out(kb, outb, sem_ob)

        @pl.when(i < NPAIRA - 1)
        def _():
            start_in(kb + 2 * NW, inb, bbb, sem_ib, sem_bb)

        return 0

    lax.fori_loop(0, NPAIRA, body, 0)

    @pl.when(has_odd)
    def _():
        k = wid + 2 * NPAIRA * NW
        wait_out(outa, sem_oa)
        wait_in(ina, bba, sem_ia, sem_ba)
        _apply_chunk(ina, outa, bba, adbuf, CH)
        start_out(k, outa, sem_oa)
        wait_out(outa, sem_oa)

    @pl.when(is_tail)
    def _():
        r0 = FULLA * CH
        wait_out(outa, sem_oa)
        pltpu.sync_copy(x_hbm.at[pl.ds(r0, TAILA), :],
                        ina.at[pl.ds(0, TAILA), :])
        pltpu.sync_copy(bidx_hbm.at[pl.ds(r0, TAILA)],
                        bba.at[pl.ds(0, TAILA)])
        _apply_chunk(ina, outa, bba, adbuf, TAILA)
        pltpu.sync_copy(outa.at[pl.ds(0, TAILA), :],
                        out_hbm.at[pl.ds(r0, TAILA), :])

    @pl.when(jnp.logical_not(has_odd) & jnp.logical_not(is_tail))
    def _():
        wait_out(outa, sem_oa)

    wait_out(outb, sem_ob)


_sc_apply = functools.partial(
    pl.kernel,
    out_type=jax.ShapeDtypeStruct((N, C), jnp.float32),
    mesh=_mesh,
    scratch_types=[
        pltpu.VMEM((CH, C), jnp.float32),
        pltpu.VMEM((CH, C), jnp.float32),
        pltpu.VMEM((CH, C), jnp.float32),
        pltpu.VMEM((CH, C), jnp.float32),
        pltpu.VMEM((BBUF,), jnp.int32),
        pltpu.VMEM((BBUF,), jnp.int32),
        pltpu.VMEM((2 * B * C,), jnp.float32),
        pltpu.VMEM((8, C), jnp.float32),
        pltpu.VMEM((B, 2 * C), jnp.float32),
        pltpu.VMEM((C,), jnp.float32),
        pltpu.VMEM((C,), jnp.float32),
        pltpu.SemaphoreType.DMA,
        pltpu.SemaphoreType.DMA,
        pltpu.SemaphoreType.DMA,
        pltpu.SemaphoreType.DMA,
        pltpu.SemaphoreType.DMA,
        pltpu.SemaphoreType.DMA,
    ],
)(_apply_body)


@jax.jit
def kernel(x, dataset_token, coors, bn_gamma, bn_beta, W, b):
    sc_sums = _sc_stats(x)

    tc_sums, sc_mlp = pl.pallas_call(
        _tc_stats_body,
        grid=(NBT,),
        in_specs=[
            pl.BlockSpec((B, CTX), lambda i: (0, 0)),
            pl.BlockSpec((2 * C, CTX), lambda i: (0, 0)),
            pl.BlockSpec((1, 2 * C), lambda i: (0, 0)),
            pl.BlockSpec((RB, C), lambda i: (i + OFF, 0)),
        ],
        out_specs=[
            pl.BlockSpec((8, C), lambda i: (0, 0)),
            pl.BlockSpec((B, 2 * C), lambda i: (0, 0)),
        ],
        out_shape=[
            jax.ShapeDtypeStruct((8, C), jnp.float32),
            jax.ShapeDtypeStruct((B, 2 * C), jnp.float32),
        ],
    )(dataset_token, W, b.reshape(1, 2 * C), x)

    bidx = coors.reshape(N)
    return _sc_apply(x, bidx, sc_sums, tc_sums, sc_mlp, bn_gamma, bn_beta)


# speedup vs baseline: 1.0373x; 1.0373x over previous
import functools

import jax
import jax.numpy as jnp
from jax import lax
from jax.experimental import pallas as pl
from jax.experimental.pallas import tpu as pltpu
from jax.experimental.pallas import tpu_sc as plsc

N, C, B, CTX = 100000, 256, 4, 256
NC, NS, L = 2, 16, 16
NW = NC * NS
CH = 120
G = C // L
BBUF = CH + L

N_SC = 42000
FULLS = N_SC // CH
NPAIRS = FULLS // NW // 2

RB = 2000
OFF = N_SC // RB
NBT = (N - N_SC) // RB

FULLA = N // CH
NPAIRA = FULLA // NW // 2
TAILA = N - FULLA * CH

_mesh = plsc.VectorSubcoreMesh(core_axis_name="c", subcore_axis_name="s",
                               num_cores=NC, num_subcores=NS)


def _wid():
    return lax.axis_index("s") * NC + lax.axis_index("c")


def _row0(k):
    return pl.multiple_of(k * CH, 8)


def _accum_carry(xbuf, carry, nrows):
    def row_body(r, c):
        out = []
        for g in range(G):
            v = xbuf[r, pl.ds(g * L, L)]
            out.append(c[g] + v)
        for g in range(G):
            v = xbuf[r, pl.ds(g * L, L)]
            out.append(c[G + g] + v * v)
        return tuple(out)

    return lax.fori_loop(0, nrows, row_body, carry)


def _accum_ref(xbuf, accbuf, nrows):
    def row_body(r, _):
        for g in range(G):
            v = xbuf[r, pl.ds(g * L, L)]
            accbuf[0, pl.ds(g * L, L)] += v
            accbuf[1, pl.ds(g * L, L)] += v * v
        return 0

    lax.fori_loop(0, nrows, row_body, 0)


def _stats_body(x_hbm, out_hbm, xa, xb, accbuf, sem_a, sem_b):
    wid = _wid()
    has_odd = wid + 2 * NPAIRS * NW < FULLS

    def start(k, buf, sem):
        pltpu.async_copy(x_hbm.at[pl.ds(_row0(k), CH), :], buf, sem)

    def wait(buf, sem):
        pltpu.make_async_copy(x_hbm.at[pl.ds(0, CH), :], buf, sem).wait()

    start(wid, xa, sem_a)
    start(wid + NW, xb, sem_b)

    def body(i, carry):
        wait(xa, sem_a)
        carry = _accum_carry(xa, carry, CH)

        @pl.when((i < NPAIRS - 1) | has_odd)
        def _():
            start(wid + (2 * i + 2) * NW, xa, sem_a)

        wait(xb, sem_b)
        carry = _accum_carry(xb, carry, CH)

        @pl.when(i < NPAIRS - 1)
        def _():
            start(wid + (2 * i + 3) * NW, xb, sem_b)

        return carry

    zero = jnp.zeros((L,), jnp.float32)
    acc = lax.fori_loop(0, NPAIRS, body, (zero,) * (2 * G))

    for g in range(G):
        accbuf[0, pl.ds(g * L, L)] = acc[g]
        accbuf[1, pl.ds(g * L, L)] = acc[G + g]

    @pl.when(has_odd)
    def _():
        wait(xa, sem_a)
        _accum_ref(xa, accbuf, CH)

    pltpu.sync_copy(accbuf.at[pl.ds(0, 1), :], out_hbm.at[pl.ds(wid, 1), :])
    pltpu.sync_copy(accbuf.at[pl.ds(1, 1), :],
                    out_hbm.at[pl.ds(NW + wid, 1), :])


_sc_stats = functools.partial(
    pl.kernel,
    out_type=jax.ShapeDtypeStruct((2 * NW, C), jnp.float32),
    mesh=_mesh,
    scratch_types=[
        pltpu.VMEM((CH, C), jnp.float32),
        pltpu.VMEM((CH, C), jnp.float32),
        pltpu.VMEM((2, C), jnp.float32),
        pltpu.SemaphoreType.DMA,
        pltpu.SemaphoreType.DMA,
    ],
)(_stats_body)


def _tc_stats_body(x_ref, out_ref):
    @pl.when(pl.program_id(0) == 0)
    def _():
        out_ref[...] = jnp.zeros_like(out_ref)

    xv = x_ref[...]
    out_ref[0:1, :] += jnp.sum(xv, axis=0, keepdims=True)
    out_ref[1:2, :] += jnp.sum(xv * xv, axis=0, keepdims=True)


RBA = 2000
NBA = N // RBA


def _tc_apply_body(sc_sums_ref, tc_sums_ref, tok_ref, w_ref, b_ref,
                   gamma_ref, beta_ref, bidx_ref, x_ref, out_ref, ad_ref):
    @pl.when(pl.program_id(0) == 0)
    def _():
        s = sc_sums_ref[...]
        t = tc_sums_ref[...]
        mean = (jnp.sum(s[0:NW, :], axis=0, keepdims=True)
                + t[0:1, :]) / N
        msq = (jnp.sum(s[NW:2 * NW, :], axis=0, keepdims=True)
               + t[1:2, :]) / N
        var = msq - mean * mean
        g = gamma_ref[...] * lax.rsqrt(var + 1e-5)
        base_shift = beta_ref[...] - mean * g
        tok = tok_ref[...]
        h = tok * (1.0 / (1.0 + jnp.exp(-tok)))
        sc = lax.dot_general(h, w_ref[...], (((1,), (1,)), ((), ())),
                             preferred_element_type=jnp.float32)
        sc = sc + b_ref[...]
        shift = sc[:, :C]
        scale = sc[:, C:]
        one_p = 1.0 + scale
        ad_ref[0:B, :] = one_p * g
        ad_ref[B:2 * B, :] = one_p * base_shift + shift

    bidx = bidx_ref[0, 0, :]
    oh = (bidx[:, None] ==
          lax.broadcasted_iota(jnp.int32, (RBA, B), 1)).astype(jnp.float32)
    a_rows = jnp.dot(oh, ad_ref[0:B, :], preferred_element_type=jnp.float32)
    d_rows = jnp.dot(oh, ad_ref[B:2 * B, :],
                     preferred_element_type=jnp.float32)
    out_ref[...] = x_ref[...] * a_rows + d_rows


@jax.jit
def kernel(x, dataset_token, coors, bn_gamma, bn_beta, W, b):
    sc_sums = _sc_stats(x)

    tc_sums = pl.pallas_call(
        _tc_stats_body,
        grid=(NBT,),
        in_specs=[pl.BlockSpec((RB, C), lambda i: (i + OFF, 0))],
        out_specs=pl.BlockSpec((8, C), lambda i: (0, 0)),
        out_shape=jax.ShapeDtypeStruct((8, C), jnp.float32),
    )(x)

    bidx3 = coors.reshape(NBA, 1, RBA)
    return pl.pallas_call(
        _tc_apply_body,
        grid=(NBA,),
        in_specs=[
            pl.BlockSpec((2 * NW, C), lambda i: (0, 0)),
            pl.BlockSpec((8, C), lambda i: (0, 0)),
            pl.BlockSpec((B, CTX), lambda i: (0, 0)),
            pl.BlockSpec((2 * C, CTX), lambda i: (0, 0)),
            pl.BlockSpec((1, 2 * C), lambda i: (0, 0)),
            pl.BlockSpec((1, C), lambda i: (0, 0)),
            pl.BlockSpec((1, C), lambda i: (0, 0)),
            pl.BlockSpec((1, 1, RBA), lambda i: (i, 0, 0)),
            pl.BlockSpec((RBA, C), lambda i: (i, 0)),
        ],
        out_specs=pl.BlockSpec((RBA, C), lambda i: (i, 0)),
        out_shape=jax.ShapeDtypeStruct((N, C), jnp.float32),
        scratch_shapes=[pltpu.VMEM((2 * B, C), jnp.float32)],
    )(sc_sums, tc_sums, dataset_token, W, b.reshape(1, 2 * C),
      bn_gamma.reshape(1, C), bn_beta.reshape(1, C), bidx3, x)


# speedup vs baseline: 1.0703x; 1.0319x over previous
import functools

import jax
import jax.numpy as jnp
from jax import lax
from jax.experimental import pallas as pl
from jax.experimental.pallas import tpu as pltpu
from jax.experimental.pallas import tpu_sc as plsc

N, C, B, CTX = 100000, 256, 4, 256
NC, NS, L = 2, 16, 16
NW = NC * NS
CH = 120
G = C // L
BBUF = CH + L

N_SC = 42000
FULLS = N_SC // CH
NPAIRS = FULLS // NW // 2

RB = 2000
OFF = N_SC // RB
NBT = (N - N_SC) // RB

FULLA = N // CH
NPAIRA = FULLA // NW // 2
TAILA = N - FULLA * CH

_mesh = plsc.VectorSubcoreMesh(core_axis_name="c", subcore_axis_name="s",
                               num_cores=NC, num_subcores=NS)


def _wid():
    return lax.axis_index("s") * NC + lax.axis_index("c")


def _row0(k):
    return pl.multiple_of(k * CH, 8)


def _accum_carry(xbuf, carry, nrows):
    def row_body(r, c):
        out = []
        for g in range(G):
            v = xbuf[r, pl.ds(g * L, L)]
            out.append(c[g] + v)
        for g in range(G):
            v = xbuf[r, pl.ds(g * L, L)]
            out.append(c[G + g] + v * v)
        return tuple(out)

    return lax.fori_loop(0, nrows, row_body, carry)


def _accum_ref(xbuf, accbuf, nrows):
    def row_body(r, _):
        for g in range(G):
            v = xbuf[r, pl.ds(g * L, L)]
            accbuf[0, pl.ds(g * L, L)] += v
            accbuf[1, pl.ds(g * L, L)] += v * v
        return 0

    lax.fori_loop(0, nrows, row_body, 0)


def _stats_body(x_hbm, out_hbm, xa, xb, accbuf, sem_a, sem_b):
    wid = _wid()
    has_odd = wid + 2 * NPAIRS * NW < FULLS

    def start(k, buf, sem):
        pltpu.async_copy(x_hbm.at[pl.ds(_row0(k), CH), :], buf, sem)

    def wait(buf, sem):
        pltpu.make_async_copy(x_hbm.at[pl.ds(0, CH), :], buf, sem).wait()

    start(wid, xa, sem_a)
    start(wid + NW, xb, sem_b)

    def body(i, carry):
        wait(xa, sem_a)
        carry = _accum_carry(xa, carry, CH)

        @pl.when((i < NPAIRS - 1) | has_odd)
        def _():
            start(wid + (2 * i + 2) * NW, xa, sem_a)

        wait(xb, sem_b)
        carry = _accum_carry(xb, carry, CH)

        @pl.when(i < NPAIRS - 1)
        def _():
            start(wid + (2 * i + 3) * NW, xb, sem_b)

        return carry

    zero = jnp.zeros((L,), jnp.float32)
    acc = lax.fori_loop(0, NPAIRS, body, (zero,) * (2 * G))

    for g in range(G):
        accbuf[0, pl.ds(g * L, L)] = acc[g]
        accbuf[1, pl.ds(g * L, L)] = acc[G + g]

    @pl.when(has_odd)
    def _():
        wait(xa, sem_a)
        _accum_ref(xa, accbuf, CH)

    pltpu.sync_copy(accbuf.at[pl.ds(0, 1), :], out_hbm.at[pl.ds(wid, 1), :])
    pltpu.sync_copy(accbuf.at[pl.ds(1, 1), :],
                    out_hbm.at[pl.ds(NW + wid, 1), :])


_sc_stats = functools.partial(
    pl.kernel,
    out_type=jax.ShapeDtypeStruct((2 * NW, C), jnp.float32),
    mesh=_mesh,
    scratch_types=[
        pltpu.VMEM((CH, C), jnp.float32),
        pltpu.VMEM((CH, C), jnp.float32),
        pltpu.VMEM((2, C), jnp.float32),
        pltpu.SemaphoreType.DMA,
        pltpu.SemaphoreType.DMA,
    ],
)(_stats_body)


def _tc_stats_body(x_ref, out_ref):
    @pl.when(pl.program_id(0) == 0)
    def _():
        out_ref[...] = jnp.zeros_like(out_ref)

    xv = x_ref[...]
    out_ref[0:1, :] += jnp.sum(xv, axis=0, keepdims=True)
    out_ref[1:2, :] += jnp.sum(xv * xv, axis=0, keepdims=True)


def _finalize_body(sc_sums_ref, tc_sums_ref, tok_ref, w_ref, b_ref,
                   gamma_ref, beta_ref, ad_ref):
    s = sc_sums_ref[...]
    t = tc_sums_ref[...]
    mean = (jnp.sum(s[0:NW, :], axis=0, keepdims=True)
            + t[0:1, :]) / N
    msq = (jnp.sum(s[NW:2 * NW, :], axis=0, keepdims=True)
           + t[1:2, :]) / N
    var = msq - mean * mean
    g = gamma_ref[...] * lax.rsqrt(var + 1e-5)
    base_shift = beta_ref[...] - mean * g
    tok = tok_ref[...]
    h = tok * (1.0 / (1.0 + jnp.exp(-tok)))
    sc = lax.dot_general(h, w_ref[...], (((1,), (1,)), ((), ())),
                         preferred_element_type=jnp.float32)
    sc = sc + b_ref[...]
    shift = sc[:, :C]
    scale = sc[:, C:]
    one_p = 1.0 + scale
    ad_ref[0:B, :] = one_p * g
    ad_ref[B:2 * B, :] = one_p * base_shift + shift


def _gather_ad(adbuf, brow):
    base = brow * C
    avs, dvs = [], []
    for g in range(G):
        avs.append(adbuf[pl.ds(base + g * L, L)])
        dvs.append(adbuf[pl.ds(base + (B * C + g * L), L)])
    return avs, dvs


def _apply_chunk(inbuf, outbuf, bbuf, adbuf, nrows):
    b0 = bbuf[pl.ds(0, L)][0]
    b1 = bbuf[pl.ds(nrows - L, L)][L - 1]

    @pl.when(b0 == b1)
    def _():
        avs, dvs = _gather_ad(adbuf, b0)

        def row_body(r, _):
            for g in range(G):
                sl = pl.ds(g * L, L)
                outbuf[r, sl] = inbuf[r, sl] * avs[g] + dvs[g]
            return 0

        lax.fori_loop(0, nrows, row_body, 0)

    @pl.when(b0 != b1)
    def _():
        def row_body(r, _):
            avs, dvs = _gather_ad(adbuf, bbuf[pl.ds(r, L)][0])
            for g in range(G):
                sl = pl.ds(g * L, L)
                outbuf[r, sl] = inbuf[r, sl] * avs[g] + dvs[g]
            return 0

        lax.fori_loop(0, nrows, row_body, 0)


def _apply_body(x_hbm, bidx_hbm, ad_hbm, out_hbm,
                ina, inb, outa, outb, bba, bbb, adbuf,
                sem_ia, sem_ib, sem_oa, sem_ob, sem_ba, sem_bb):
    wid = _wid()
    has_odd = wid + 2 * NPAIRA * NW < FULLA
    is_tail = wid == NW - 1
    pltpu.sync_copy(ad_hbm, adbuf)

    def start_in(k, buf, bbuf, sem, bsem):
        r0 = _row0(k)
        pltpu.async_copy(x_hbm.at[pl.ds(r0, CH), :], buf, sem)
        pltpu.async_copy(bidx_hbm.at[pl.ds(r0, CH)],
                         bbuf.at[pl.ds(0, CH)], bsem)

    def wait_in(buf, bbuf, sem, bsem):
        pltpu.make_async_copy(x_hbm.at[pl.ds(0, CH), :], buf, sem).wait()
        pltpu.make_async_copy(bidx_hbm.at[pl.ds(0, CH)],
                              bbuf.at[pl.ds(0, CH)], bsem).wait()

    def start_out(k, buf, sem):
        pltpu.async_copy(buf, out_hbm.at[pl.ds(_row0(k), CH), :], sem)

    def wait_out(buf, sem):
        pltpu.make_async_copy(buf, out_hbm.at[pl.ds(0, CH), :], sem).wait()

    start_in(wid, ina, bba, sem_ia, sem_ba)
    start_in(wid + NW, inb, bbb, sem_ib, sem_bb)

    def body(i, _):
        ka = wid + 2 * i * NW

        @pl.when(i > 0)
        def _():
            wait_out(outa, sem_oa)

        wait_in(ina, bba, sem_ia, sem_ba)
        _apply_chunk(ina, outa, bba, adbuf, CH)
        start_out(ka, outa, sem_oa)

        @pl.when((i < NPAIRA - 1) | has_odd)
        def _():
            start_in(ka + 2 * NW, ina, bba, sem_ia, sem_ba)

        kb = ka + NW

        @pl.when(i > 0)
        def _():
            wait_out(outb, sem_ob)

        wait_in(inb, bbb, sem_ib, sem_bb)
        _apply_chunk(inb, outb, bbb, adbuf, CH)
        start_out(kb, outb, sem_ob)

        @pl.when(i < NPAIRA - 1)
        def _():
            start_in(kb + 2 * NW, inb, bbb, sem_ib, sem_bb)

        return 0

    lax.fori_loop(0, NPAIRA, body, 0)

    @pl.when(has_odd)
    def _():
        k = wid + 2 * NPAIRA * NW
        wait_out(outa, sem_oa)
        wait_in(ina, bba, sem_ia, sem_ba)
        _apply_chunk(ina, outa, bba, adbuf, CH)
        start_out(k, outa, sem_oa)
        wait_out(outa, sem_oa)

    @pl.when(is_tail)
    def _():
        r0 = FULLA * CH
        wait_out(outa, sem_oa)
        pltpu.sync_copy(x_hbm.at[pl.ds(r0, TAILA), :],
                        ina.at[pl.ds(0, TAILA), :])
        pltpu.sync_copy(bidx_hbm.at[pl.ds(r0, TAILA)],
                        bba.at[pl.ds(0, TAILA)])
        _apply_chunk(ina, outa, bba, adbuf, TAILA)
        pltpu.sync_copy(outa.at[pl.ds(0, TAILA), :],
                        out_hbm.at[pl.ds(r0, TAILA), :])

    @pl.when(jnp.logical_not(has_odd) & jnp.logical_not(is_tail))
    def _():
        wait_out(outa, sem_oa)

    wait_out(outb, sem_ob)


_sc_apply = functools.partial(
    pl.kernel,
    out_type=jax.ShapeDtypeStruct((N, C), jnp.float32),
    mesh=_mesh,
    scratch_types=[
        pltpu.VMEM((CH, C), jnp.float32),
        pltpu.VMEM((CH, C), jnp.float32),
        pltpu.VMEM((CH, C), jnp.float32),
        pltpu.VMEM((CH, C), jnp.float32),
        pltpu.VMEM((BBUF,), jnp.int32),
        pltpu.VMEM((BBUF,), jnp.int32),
        pltpu.VMEM((2 * B * C,), jnp.float32),
        pltpu.SemaphoreType.DMA,
        pltpu.SemaphoreType.DMA,
        pltpu.SemaphoreType.DMA,
        pltpu.SemaphoreType.DMA,
        pltpu.SemaphoreType.DMA,
        pltpu.SemaphoreType.DMA,
    ],
)(_apply_body)


@jax.jit
def kernel(x, dataset_token, coors, bn_gamma, bn_beta, W, b):
    sc_sums = _sc_stats(x)

    tc_sums = pl.pallas_call(
        _tc_stats_body,
        grid=(NBT,),
        in_specs=[pl.BlockSpec((RB, C), lambda i: (i + OFF, 0))],
        out_specs=pl.BlockSpec((8, C), lambda i: (0, 0)),
        out_shape=jax.ShapeDtypeStruct((8, C), jnp.float32),
    )(x)

    ad = pl.pallas_call(
        _finalize_body,
        in_specs=[
            pl.BlockSpec((2 * NW, C), lambda: (0, 0)),
            pl.BlockSpec((8, C), lambda: (0, 0)),
            pl.BlockSpec((B, CTX), lambda: (0, 0)),
            pl.BlockSpec((2 * C, CTX), lambda: (0, 0)),
            pl.BlockSpec((1, 2 * C), lambda: (0, 0)),
            pl.BlockSpec((1, C), lambda: (0, 0)),
            pl.BlockSpec((1, C), lambda: (0, 0)),
        ],
        out_specs=pl.BlockSpec((2 * B, C), lambda: (0, 0)),
        out_shape=jax.ShapeDtypeStruct((2 * B, C), jnp.float32),
    )(sc_sums, tc_sums, dataset_token, W, b.reshape(1, 2 * C),
      bn_gamma.reshape(1, C), bn_beta.reshape(1, C))

    bidx = coors.reshape(N)
    return _sc_apply(x, bidx, ad.reshape(2 * B * C))
